# baseline (speedup 1.0000x reference)
import dataclasses
import functools

import jax
import jax.numpy as jnp
from jax import lax
from jax.experimental import pallas as pl
from jax.experimental.pallas import tpu as pltpu
from jax.experimental.pallas import tpu_sc as plsc

_VOCAB = 1000000
_EMB = 64
_B = 4096
_L = 200

_NC, _NS = 2, 16
_NW = _NC * _NS
_ROWS_W = _B // _NW
_PERW = _ROWS_W * _L
_CHA = 104
_CHB = _L - _CHA
_STRIDE = 208

_SC_PARAMS = pltpu.CompilerParams(use_tc_tiling_on_sc=False)
if "needs_layout_passes" in pltpu.CompilerParams.__dataclass_fields__:
    _SC_PARAMS = dataclasses.replace(_SC_PARAMS, needs_layout_passes=False)

_SC_PARAMS_COMPACT = pltpu.CompilerParams()
if "needs_layout_passes" in pltpu.CompilerParams.__dataclass_fields__:
    _SC_PARAMS_COMPACT = dataclasses.replace(
        _SC_PARAMS_COMPACT, needs_layout_passes=False)


@functools.partial(
    pl.kernel,
    out_type=jax.ShapeDtypeStruct((_NW, _L, _ROWS_W), jnp.int32),
    mesh=plsc.VectorSubcoreMesh(core_axis_name="c", subcore_axis_name="s"),
    compiler_params=_SC_PARAMS_COMPACT,
    scratch_types=[
        pltpu.VMEM((_L, _ROWS_W), jnp.int32),
    ],
)
def _sc_stage_idx(xt_hbm, o_hbm, slab_v):
    wid = lax.axis_index("s") * _NC + lax.axis_index("c")
    pltpu.sync_copy(xt_hbm.at[:, pl.ds(wid * _ROWS_W, _ROWS_W)], slab_v)
    pltpu.sync_copy(slab_v, o_hbm.at[wid])


@functools.partial(
    pl.kernel,
    out_type=jax.ShapeDtypeStruct((_B,), jnp.float32),
    mesh=plsc.VectorSubcoreMesh(core_axis_name="c", subcore_axis_name="s"),
    compiler_params=_SC_PARAMS,
    scratch_types=[
        pltpu.VMEM((_L + 8, _ROWS_W), jnp.int32),
        pltpu.VMEM((_ROWS_W * _STRIDE,), jnp.int32),
        pltpu.VMEM((2, _L, _EMB), jnp.float32),
        pltpu.VMEM((80,), jnp.float32),
        pltpu.VMEM((_ROWS_W,), jnp.float32),
        pltpu.SemaphoreType.DMA,
        pltpu.SemaphoreType.DMA,
    ],
)
def _sc_embed_dot(tbl_hbm, xl_hbm, wb_hbm, o_hbm,
                  idxt_v, idx_v, rows_v, wb_v, outv, sem0, sem1):
    wid = lax.axis_index("s") * _NC + lax.axis_index("c")
    pltpu.sync_copy(wb_hbm, wb_v)
    pltpu.sync_copy(xl_hbm.at[wid], idxt_v.at[pl.ds(0, _L), :])

    lanes16 = lax.iota(jnp.int32, 16)
    jvecs = [lanes16 + 16 * k for k in range(13)]

    def reorder(r, carry):
        rvec = jnp.zeros((16,), jnp.int32) + r
        for k in range(13):
            v = plsc.load_gather(idxt_v, [jvecs[k], rvec])
            idx_v[pl.ds(r * _STRIDE + 16 * k, 16)] = v
        return carry

    lax.fori_loop(0, _ROWS_W, reorder, 0)

    sems = (sem0, sem1)

    def fire(r, buf):
        off = r * _STRIDE
        pltpu.async_copy(
            tbl_hbm.at[idx_v.at[pl.ds(off, _CHA)]],
            rows_v.at[buf, pl.ds(0, _CHA)], sems[buf])
        pltpu.async_copy(
            tbl_hbm.at[idx_v.at[pl.ds(off + _CHA, _CHB)]],
            rows_v.at[buf, pl.ds(_CHA, _CHB)], sems[buf])

    def drain(buf):
        pltpu.make_async_copy(
            tbl_hbm.at[pl.ds(0, _CHA), :],
            rows_v.at[buf, pl.ds(0, _CHA)], sems[buf]).wait()
        pltpu.make_async_copy(
            tbl_hbm.at[pl.ds(0, _CHB), :],
            rows_v.at[buf, pl.ds(_CHA, _CHB)], sems[buf]).wait()

    wv = [wb_v[pl.ds(16 * i, 16)] for i in range(4)]
    bvec = wb_v[pl.ds(64, 16)]
    lanes = lax.iota(jnp.int32, 16)
    zero16 = jnp.zeros((16,), jnp.float32)

    def row_sum(buf, r):
        def body(k, accs):
            a0, a1, a2, a3 = accs
            return (a0 + rows_v[buf, k, pl.ds(0, 16)],
                    a1 + rows_v[buf, k, pl.ds(16, 16)],
                    a2 + rows_v[buf, k, pl.ds(32, 16)],
                    a3 + rows_v[buf, k, pl.ds(48, 16)])
        a0, a1, a2, a3 = lax.fori_loop(
            0, _L, body, (zero16, zero16, zero16, zero16), unroll=4)
        m = a0 * wv[0] + a1 * wv[1] + a2 * wv[2] + a3 * wv[3]
        return lax.reduce_sum_p.bind(m, axes=(0,))

    fire(0, 0)
    fire(1, 1)

    for g in range(_ROWS_W // 16):

        def pair(rp, res, g=g):
            r0 = g * 16 + 2 * rp
            drain(0)
            s0 = row_sum(0, r0)

            @pl.when(r0 + 2 < _ROWS_W)
            def _():
                fire(r0 + 2, 0)

            drain(1)
            s1 = row_sum(1, r0 + 1)

            @pl.when(r0 + 3 < _ROWS_W)
            def _():
                fire(r0 + 3, 1)

            res = jnp.where(lanes == 2 * rp, s0, res)
            res = jnp.where(lanes == 2 * rp + 1, s1, res)
            return res

        res = lax.fori_loop(0, 8, pair, zero16)
        outv[pl.ds(g * 16, 16)] = res + bvec

    pltpu.sync_copy(outv, o_hbm.at[pl.ds(wid * _ROWS_W, _ROWS_W)])


def kernel(x, table, W, b):
    ws = W.astype(jnp.float32).reshape(_EMB) * (1.0 / _L)
    wb = jnp.concatenate([ws, jnp.broadcast_to(b.astype(jnp.float32), (16,))])
    xl3 = _sc_stage_idx(x.astype(jnp.int32).T)
    return _sc_embed_dot(table, xl3, wb)


# speedup vs baseline: 1.2847x; 1.2847x over previous
import dataclasses
import functools

import jax
import jax.numpy as jnp
from jax import lax
from jax.experimental import pallas as pl
from jax.experimental.pallas import tpu as pltpu
from jax.experimental.pallas import tpu_sc as plsc

_VOCAB = 1000000
_EMB = 64
_B = 4096
_L = 200

_BLK_R = 32768

_NC, _NS = 2, 16
_NW = _NC * _NS
_ROWS_W = _B // _NW
_PERW = _ROWS_W * _L
_CH = 128
_NCHUNK = _PERW // _CH
_RING = 8
_GROUPS = _ROWS_W // 16
_GSTRIDE = 16 * _L
_JCH = _GSTRIDE // _CH


def _tc_body(ta_ref, tb_ref, w_ref, b_ref, o_ref):
    half = _BLK_R // 2
    pa = ta_ref[...] * w_ref[...]
    o_ref[pl.ds(0, half)] = jnp.sum(pa.T, axis=0) + b_ref[0, 0]
    pb = tb_ref[...] * w_ref[...]
    o_ref[pl.ds(half, half)] = jnp.sum(pb.T, axis=0) + b_ref[0, 0]


def _table_times_w(table, wrow, brow):
    half = _BLK_R // 2
    grid = pl.cdiv(_VOCAB, _BLK_R)
    return pl.pallas_call(
        _tc_body,
        grid=(grid,),
        in_specs=[
            pl.BlockSpec((half, _EMB), lambda i: (2 * i, 0)),
            pl.BlockSpec((half, _EMB), lambda i: (2 * i + 1, 0)),
            pl.BlockSpec((1, _EMB), lambda i: (0, 0)),
            pl.BlockSpec((1, 1), lambda i: (0, 0)),
        ],
        out_specs=pl.BlockSpec((_BLK_R,), lambda i: (i,)),
        out_shape=jax.ShapeDtypeStruct((_VOCAB,), jnp.float32),
    )(table, table, wrow, brow)


_SC_PARAMS = pltpu.CompilerParams()
if "needs_layout_passes" in pltpu.CompilerParams.__dataclass_fields__:
    _SC_PARAMS = dataclasses.replace(_SC_PARAMS, needs_layout_passes=False)


@functools.partial(
    pl.kernel,
    out_type=jax.ShapeDtypeStruct((_B,), jnp.float32),
    mesh=plsc.VectorSubcoreMesh(core_axis_name="c", subcore_axis_name="s"),
    compiler_params=_SC_PARAMS,
    scratch_types=[
        pltpu.VMEM((_NCHUNK, _CH), jnp.int32),
        pltpu.VMEM((_PERW,), jnp.float32),
        pltpu.VMEM((_ROWS_W,), jnp.float32),
        pltpu.SemaphoreType.DMA,
    ],
)
def _sc_gather_sum(t_hbm, xr_hbm, o_hbm, idx_v, vals_v, outv, sem):
    wid = lax.axis_index("s") * _NC + lax.axis_index("c")
    pltpu.sync_copy(xr_hbm.at[wid], idx_v)

    for p in range(_RING):
        pltpu.async_copy(
            t_hbm.at[idx_v.at[p]], vals_v.at[pl.ds(p * _CH, _CH)], sem)

    @pl.loop(_RING, _NCHUNK)
    def _(c):
        pltpu.make_async_copy(
            t_hbm.at[pl.ds(0, _CH)], vals_v.at[pl.ds(0, _CH)], sem).wait()
        pltpu.async_copy(
            t_hbm.at[idx_v.at[c]], vals_v.at[pl.ds(c * _CH, _CH)], sem)

    for p in range(_RING):
        pltpu.make_async_copy(
            t_hbm.at[pl.ds(0, _CH)], vals_v.at[pl.ds(0, _CH)], sem).wait()

    lanes = lax.iota(jnp.int32, 16)
    for g in range(_GROUPS):
        base = (lanes + g * 16) * _L

        def body(j, acc, base=base):
            return acc + plsc.load_gather(vals_v, [base + j])

        acc = lax.fori_loop(0, _L, body, jnp.zeros((16,), jnp.float32),
                            unroll=8)
        outv[pl.ds(g * 16, 16)] = acc

    pltpu.sync_copy(outv, o_hbm.at[pl.ds(wid * _ROWS_W, _ROWS_W)])


def kernel(x, table, W, b):
    wrow = (W.astype(jnp.float32) * (1.0 / _L)).reshape(1, _EMB)
    brow = (b.astype(jnp.float32) * (1.0 / _L)).reshape(1, 1)
    t = _table_times_w(table, wrow, brow)
    xr3 = x.astype(jnp.int32).reshape(_NW, _NCHUNK, _CH)
    return _sc_gather_sum(t, xr3)
